# baseline (speedup 1.0000x reference)
import functools

import jax
import jax.numpy as jnp
from jax import lax
from jax.experimental import pallas as pl
from jax.experimental.pallas import tpu as pltpu
from jax.experimental.pallas import tpu_sc as plsc

B, V, C, P, D = 64, 10, 8, 3, 128
NCODE = B * V * C
NPAR = NCODE * P
NK = B * C * P
N_IDX = 2 * NCODE + 2 * NPAR + NCODE + 2 * NK
NWORK = 32
IPW = N_IDX // NWORK

ROW_EMB_C = 0
ROW_EMB_P = NCODE
ROW_CPAR = 2 * NCODE
ROW_PPAR = ROW_CPAR + NPAR
ROW_DRUG = ROW_PPAR + NPAR
ROW_KC = ROW_DRUG + NCODE
ROW_KP = ROW_KC + NK

_PLAN = [
    (0, 0, 0, 128, ROW_EMB_C, 160, 0),
    (0, 1, 0, 32, ROW_EMB_C, 160, 128),
    (1, 1, 32, 96, ROW_EMB_P, 160, 0),
    (1, 2, 0, 64, ROW_EMB_P, 160, 96),
    (2, 2, 64, 64, ROW_CPAR, 480, 0),
    (2, 3, 0, 128, ROW_CPAR, 480, 64),
    (2, 4, 0, 128, ROW_CPAR, 480, 192),
    (2, 5, 0, 128, ROW_CPAR, 480, 320),
    (2, 6, 0, 32, ROW_CPAR, 480, 448),
    (3, 6, 32, 96, ROW_PPAR, 480, 0),
    (3, 7, 0, 128, ROW_PPAR, 480, 96),
    (3, 8, 0, 128, ROW_PPAR, 480, 224),
    (3, 9, 0, 128, ROW_PPAR, 480, 352),
    (4, 10, 0, 128, ROW_DRUG, 160, 0),
    (4, 11, 0, 32, ROW_DRUG, 160, 128),
    (2, 11, 32, 48, ROW_KC, 48, 0),
    (3, 11, 80, 48, ROW_KP, 48, 0),
]

GRID1 = 8
EBLK = NCODE // GRID1
VBLK = EBLK // C


def _dotT(a, b):
    return lax.dot_general(a, b, (((1,), (1,)), ((), ())),
                           preferred_element_type=jnp.float32)


@functools.cache
def _make_sc_gather():
    @functools.partial(
        pl.kernel,
        mesh=plsc.VectorSubcoreMesh(core_axis_name="c", subcore_axis_name="s"),
        out_type=jax.ShapeDtypeStruct((N_IDX, D), jnp.float32),
        scratch_types=[
            pltpu.VMEM((IPW // 128, 128), jnp.int32),
            pltpu.VMEM((128, D), jnp.float32),
            pltpu.VMEM((128, D), jnp.float32),
            pltpu.VMEM((128, D), jnp.float32),
            pltpu.VMEM((128, D), jnp.float32),
            pltpu.SemaphoreType.DMA,
            pltpu.SemaphoreType.DMA,
            pltpu.SemaphoreType.DMA,
            pltpu.SemaphoreType.DMA,
            pltpu.SemaphoreType.DMA,
            pltpu.SemaphoreType.DMA,
            pltpu.SemaphoreType.DMA,
            pltpu.SemaphoreType.DMA,
        ],
    )
    def _sc_gather(t0, t1, t2, t3, t4, idx_hbm, out_hbm,
                   idx_v, rowA, rowB, rowC, rowD,
                   gsA, gsB, gsC, gsD, wsA, wsB, wsC, wsD):
        wid = lax.axis_index("s") * 2 + lax.axis_index("c")
        tbls = (t0, t1, t2, t3, t4)
        bufs = (rowA, rowB, rowC, rowD)
        gsems = (gsA, gsB, gsC, gsD)
        wsems = (wsA, wsB, wsC, wsD)
        pltpu.sync_copy(idx_hbm.at[wid], idx_v)
        NP = len(_PLAN)

        def fire(k):
            t, r, off, n, base, per_w, rel = _PLAN[k]
            return pltpu.async_copy(
                tbls[t].at[idx_v.at[r, pl.ds(off, n)]],
                bufs[k % 4].at[pl.ds(0, n)], gsems[k % 4])

        gath = {}
        writ = {}
        for k in range(3):
            gath[k] = fire(k)
        for k in range(NP):
            if k + 3 < NP:
                if k - 1 >= 0:
                    writ[k - 1].wait()
                gath[k + 3] = fire(k + 3)
            elif k - 1 >= 0:
                writ[k - 1].wait()
            gath[k].wait()
            t, r, off, n, base, per_w, rel = _PLAN[k]
            writ[k] = pltpu.async_copy(
                bufs[k % 4].at[pl.ds(0, n)],
                out_hbm.at[pl.ds(base + wid * per_w + rel, n)],
                wsems[k % 4])
        writ[NP - 1].wait()
    return _sc_gather


def _attn_slab(emb, p0, p1, p2, W1a, W1b, b1, w2):
    Ha = _dotT(emb, W1a) + b1
    hs = jnp.tanh(Ha + _dotT(emb, W1b))
    ss = _dotT(hs, w2)
    scs = []
    for p in (p0, p1, p2):
        hj = jnp.tanh(Ha + _dotT(p, W1b))
        scs.append(_dotT(hj, w2))
    m = jnp.maximum(jnp.maximum(scs[0], scs[1]), jnp.maximum(scs[2], ss))
    es = jnp.exp(ss - m)
    num = es * emb
    den = es
    for p, s in zip((p0, p1, p2), scs):
        e = jnp.exp(s - m)
        num = num + e * p
        den = den + e
    ce = num / den
    return jnp.sum(ce.reshape(VBLK, C, D), axis=1)


def _attn_body(ec_ref, ep_ref, cp0_ref, cp1_ref, cp2_ref,
               pp0_ref, pp1_ref, pp2_ref, drug_ref,
               W1_ref, b1_ref, W2_ref, vc_ref, vp_ref, vd_ref):
    W1 = W1_ref[...]
    W1a = W1[:, :D]
    W1b = W1[:, D:]
    b1 = b1_ref[...]
    w2 = W2_ref[...]
    vc_ref[...] = _attn_slab(ec_ref[...], cp0_ref[...], cp1_ref[...],
                             cp2_ref[...], W1a, W1b, b1, w2)
    vp_ref[...] = _attn_slab(ep_ref[...], pp0_ref[...], pp1_ref[...],
                             pp2_ref[...], W1a, W1b, b1, w2)
    vd_ref[...] = jnp.sum(drug_ref[...].reshape(VBLK, C, D), axis=1)


_TC1_IN_SPECS = [
    pl.BlockSpec((EBLK, D), lambda i: (i, 0)),
    pl.BlockSpec((EBLK, D), lambda i: (ROW_EMB_P // EBLK + i, 0)),
] + [
    pl.BlockSpec((EBLK, D),
                 lambda i, j=j: (ROW_CPAR // EBLK + j * GRID1 + i, 0))
    for j in range(P)
] + [
    pl.BlockSpec((EBLK, D),
                 lambda i, j=j: (ROW_PPAR // EBLK + j * GRID1 + i, 0))
    for j in range(P)
] + [
    pl.BlockSpec((EBLK, D), lambda i: (ROW_DRUG // EBLK + i, 0)),
    pl.BlockSpec((D, 2 * D), lambda i: (0, 0)),
    pl.BlockSpec((1, D), lambda i: (0, 0)),
    pl.BlockSpec((1, D), lambda i: (0, 0)),
]
_TC1_OUT_SPECS = [
    pl.BlockSpec((VBLK, D), lambda i: (i, 0)),
    pl.BlockSpec((VBLK, D), lambda i: (i, 0)),
    pl.BlockSpec((VBLK, D), lambda i: (i, 0)),
]
_TC1_OUT_SHAPE = [
    jax.ShapeDtypeStruct((B * V, D), jnp.float32),
    jax.ShapeDtypeStruct((B * V, D), jnp.float32),
    jax.ShapeDtypeStruct((B * V, D), jnp.float32),
]


def _seq_body(vc_ref, vp_ref, vd_ref, k_ref,
              Wi_c, Wh_c, bi_c, bh_c, Wi_p, Wh_p, bi_p, bh_p,
              Wi_d, Wh_d, bi_d, bh_d, Kc_ref, Kp_ref, Wfc_ref, bfc_ref,
              out_ref):
    def gru(x, Wi_r, Wh_r, bi_r, bh_r):
        Wi = Wi_r[...]
        Wh = Wh_r[...]
        gi = _dotT(x, Wi) + bi_r[...]
        h = jnp.zeros((B, D), jnp.float32)
        bh = bh_r[...]
        for t in range(V):
            git = gi[t * B:(t + 1) * B]
            gh = _dotT(h, Wh) + bh
            r = jax.nn.sigmoid(git[:, :D] + gh[:, :D])
            z = jax.nn.sigmoid(git[:, D:2 * D] + gh[:, D:2 * D])
            n = jnp.tanh(git[:, 2 * D:] + r * gh[:, 2 * D:])
            h = (1.0 - z) * n + z * h
        return h

    h_c = gru(vc_ref[...], Wi_c, Wh_c, bi_c, bh_c)
    h_p = gru(vp_ref[...], Wi_p, Wh_p, bi_p, bh_p)
    h_d = gru(vd_ref[...], Wi_d, Wh_d, bi_d, bh_d)
    tmp = h_c + h_p + h_d

    krows = k_ref[...]

    def knowledge(rows, K_r):
        kp = _dotT(rows, K_r[...])
        kp3 = kp.reshape(B, C * P, D)
        w = jnp.sum(kp3 * tmp.reshape(B, 1, D), axis=2)
        m = jnp.max(w, axis=1, keepdims=True)
        e = jnp.exp(w - m)
        a = e / jnp.sum(e, axis=1, keepdims=True)
        return jnp.sum(a[:, :, None] * kp3, axis=1)

    k_c = knowledge(krows[:NK], Kc_ref)
    k_p = knowledge(krows[NK:], Kp_ref)
    patient = jnp.concatenate([h_c, h_p, h_d, k_c, k_p], axis=1)
    out_ref[...] = _dotT(patient, Wfc_ref[...]) + bfc_ref[...]


_TC2_IN_SPECS = [
    pl.BlockSpec((B * V, D), lambda i: (0, 0)),
    pl.BlockSpec((B * V, D), lambda i: (0, 0)),
    pl.BlockSpec((B * V, D), lambda i: (0, 0)),
    pl.BlockSpec((2 * NK, D), lambda i: (ROW_KC // (2 * NK), 0)),
] + [pl.BlockSpec((3 * D, D), lambda i: (0, 0)),
     pl.BlockSpec((3 * D, D), lambda i: (0, 0)),
     pl.BlockSpec((1, 3 * D), lambda i: (0, 0)),
     pl.BlockSpec((1, 3 * D), lambda i: (0, 0)),
     ] * 3 + [
    pl.BlockSpec((D, D), lambda i: (0, 0)),
    pl.BlockSpec((D, D), lambda i: (0, 0)),
    pl.BlockSpec((D, 5 * D), lambda i: (0, 0)),
    pl.BlockSpec((1, D), lambda i: (0, 0)),
]
_TC2_OUT_SPECS = pl.BlockSpec((B, D), lambda i: (0, 0))
_TC2_OUT_SHAPE = jax.ShapeDtypeStruct((B, D), jnp.float32)


def _flat_indices(cond_codes, cond_parents, proc_codes, proc_parents,
                  drug_codes, cond_last_parents, proc_last_parents):
    s1 = cond_codes.transpose(1, 0, 2).reshape(NWORK, -1)
    s2 = proc_codes.transpose(1, 0, 2).reshape(NWORK, -1)
    s3 = cond_parents.transpose(3, 1, 0, 2).reshape(NWORK, -1)
    s4 = proc_parents.transpose(3, 1, 0, 2).reshape(NWORK, -1)
    s5 = drug_codes.transpose(1, 0, 2).reshape(NWORK, -1)
    s6 = cond_last_parents.reshape(NWORK, -1)
    s7 = proc_last_parents.reshape(NWORK, -1)
    idx = jnp.concatenate([s1, s2, s3, s4, s5, s6, s7], axis=1)
    return idx.astype(jnp.int32).reshape(NWORK, IPW // 128, 128)


def kernel(cond_codes, cond_parents, proc_codes, proc_parents, drug_codes,
           cond_last_parents, proc_last_parents, E_cond, E_cond_parent,
           E_proc, E_proc_parent, E_drug, W1, b1, W2, K_cond, K_proc,
           Wi_cond, Wh_cond, bi_cond, bh_cond, Wi_proc, Wh_proc, bi_proc,
           bh_proc, Wi_drug, Wh_drug, bi_drug, bh_drug, W_fc, b_fc):
    idx3 = _flat_indices(cond_codes, cond_parents, proc_codes, proc_parents,
                         drug_codes, cond_last_parents, proc_last_parents)
    G = _make_sc_gather()(E_cond, E_proc, E_cond_parent, E_proc_parent,
                          E_drug, idx3)

    vc, vp, vd = pl.pallas_call(
        _attn_body,
        grid=(GRID1,),
        in_specs=_TC1_IN_SPECS,
        out_specs=_TC1_OUT_SPECS,
        out_shape=_TC1_OUT_SHAPE,
    )(G, G, G, G, G, G, G, G, G, W1, b1.reshape(1, D), W2)

    out = pl.pallas_call(
        _seq_body,
        grid=(1,),
        in_specs=_TC2_IN_SPECS,
        out_specs=_TC2_OUT_SPECS,
        out_shape=_TC2_OUT_SHAPE,
    )(vc, vp, vd, G,
      Wi_cond, Wh_cond, bi_cond.reshape(1, 3 * D), bh_cond.reshape(1, 3 * D),
      Wi_proc, Wh_proc, bi_proc.reshape(1, 3 * D), bh_proc.reshape(1, 3 * D),
      Wi_drug, Wh_drug, bi_drug.reshape(1, 3 * D), bh_drug.reshape(1, 3 * D),
      K_cond, K_proc, W_fc, b_fc.reshape(1, D))
    return out


# speedup vs baseline: 1.0369x; 1.0369x over previous
import functools

import jax
import jax.numpy as jnp
from jax import lax
from jax.experimental import pallas as pl
from jax.experimental.pallas import tpu as pltpu
from jax.experimental.pallas import tpu_sc as plsc

B, V, C, P, D = 64, 10, 8, 3, 128
NCODE = B * V * C
NPAR = NCODE * P
NK = B * C * P
N_IDX = 2 * NCODE + 2 * NPAR + NCODE + 2 * NK
NWORK = 32
IPW = N_IDX // NWORK

ROW_EMB_C = 0
ROW_EMB_P = NCODE
ROW_CPAR = 2 * NCODE
ROW_PPAR = ROW_CPAR + NPAR
ROW_DRUG = ROW_PPAR + NPAR
ROW_KC = ROW_DRUG + NCODE
ROW_KP = ROW_KC + NK

_PLAN = [
    (0, 0, 0, 128, ROW_EMB_C, 160, 0),
    (0, 1, 0, 32, ROW_EMB_C, 160, 128),
    (1, 1, 32, 96, ROW_EMB_P, 160, 0),
    (1, 2, 0, 64, ROW_EMB_P, 160, 96),
    (2, 2, 64, 64, ROW_CPAR, 480, 0),
    (2, 3, 0, 128, ROW_CPAR, 480, 64),
    (2, 4, 0, 128, ROW_CPAR, 480, 192),
    (2, 5, 0, 128, ROW_CPAR, 480, 320),
    (2, 6, 0, 32, ROW_CPAR, 480, 448),
    (3, 6, 32, 96, ROW_PPAR, 480, 0),
    (3, 7, 0, 128, ROW_PPAR, 480, 96),
    (3, 8, 0, 128, ROW_PPAR, 480, 224),
    (3, 9, 0, 128, ROW_PPAR, 480, 352),
    (4, 10, 0, 128, ROW_DRUG, 160, 0),
    (4, 11, 0, 32, ROW_DRUG, 160, 128),
    (2, 11, 32, 48, ROW_KC, 48, 0),
    (3, 11, 80, 48, ROW_KP, 48, 0),
]

GRID1 = 4
EBLK = NCODE // GRID1
VBLK = EBLK // C


def _dotT(a, b):
    return lax.dot_general(a, b, (((1,), (1,)), ((), ())),
                           preferred_element_type=jnp.float32)


@functools.cache
def _make_sc_gather():
    @functools.partial(
        pl.kernel,
        mesh=plsc.VectorSubcoreMesh(core_axis_name="c", subcore_axis_name="s"),
        out_type=jax.ShapeDtypeStruct((N_IDX, D), jnp.float32),
        scratch_types=[
            pltpu.VMEM((IPW // 128, 128), jnp.int32),
            pltpu.VMEM((128, D), jnp.float32),
            pltpu.VMEM((128, D), jnp.float32),
            pltpu.VMEM((128, D), jnp.float32),
            pltpu.VMEM((128, D), jnp.float32),
            pltpu.SemaphoreType.DMA,
            pltpu.SemaphoreType.DMA,
            pltpu.SemaphoreType.DMA,
            pltpu.SemaphoreType.DMA,
            pltpu.SemaphoreType.DMA,
            pltpu.SemaphoreType.DMA,
            pltpu.SemaphoreType.DMA,
            pltpu.SemaphoreType.DMA,
        ],
    )
    def _sc_gather(t0, t1, t2, t3, t4, idx_hbm, out_hbm,
                   idx_v, rowA, rowB, rowC, rowD,
                   gsA, gsB, gsC, gsD, wsA, wsB, wsC, wsD):
        wid = lax.axis_index("s") * 2 + lax.axis_index("c")
        tbls = (t0, t1, t2, t3, t4)
        bufs = (rowA, rowB, rowC, rowD)
        gsems = (gsA, gsB, gsC, gsD)
        wsems = (wsA, wsB, wsC, wsD)
        pltpu.sync_copy(idx_hbm.at[wid], idx_v)
        NP = len(_PLAN)

        def fire(k):
            t, r, off, n, base, per_w, rel = _PLAN[k]
            return pltpu.async_copy(
                tbls[t].at[idx_v.at[r, pl.ds(off, n)]],
                bufs[k % 4].at[pl.ds(0, n)], gsems[k % 4])

        gath = {}
        writ = {}
        for k in range(3):
            gath[k] = fire(k)
        for k in range(NP):
            if k + 3 < NP:
                if k - 1 >= 0:
                    writ[k - 1].wait()
                gath[k + 3] = fire(k + 3)
            elif k - 1 >= 0:
                writ[k - 1].wait()
            gath[k].wait()
            t, r, off, n, base, per_w, rel = _PLAN[k]
            writ[k] = pltpu.async_copy(
                bufs[k % 4].at[pl.ds(0, n)],
                out_hbm.at[pl.ds(base + wid * per_w + rel, n)],
                wsems[k % 4])
        writ[NP - 1].wait()
    return _sc_gather


def _attn_slab(emb, p0, p1, p2, W1a, W1b, b1, w2):
    Ha = _dotT(emb, W1a) + b1
    hs = jnp.tanh(Ha + _dotT(emb, W1b))
    ss = _dotT(hs, w2)
    scs = []
    for p in (p0, p1, p2):
        hj = jnp.tanh(Ha + _dotT(p, W1b))
        scs.append(_dotT(hj, w2))
    m = jnp.maximum(jnp.maximum(scs[0], scs[1]), jnp.maximum(scs[2], ss))
    es = jnp.exp(ss - m)
    num = es * emb
    den = es
    for p, s in zip((p0, p1, p2), scs):
        e = jnp.exp(s - m)
        num = num + e * p
        den = den + e
    ce = num / den
    return jnp.sum(ce.reshape(VBLK, C, D), axis=1)


def _attn_body(ec_ref, ep_ref, cp0_ref, cp1_ref, cp2_ref,
               pp0_ref, pp1_ref, pp2_ref, drug_ref,
               W1_ref, b1_ref, W2_ref, vc_ref, vp_ref, vd_ref):
    W1 = W1_ref[...]
    W1a = W1[:, :D]
    W1b = W1[:, D:]
    b1 = b1_ref[...]
    w2 = W2_ref[...]
    vc_ref[...] = _attn_slab(ec_ref[...], cp0_ref[...], cp1_ref[...],
                             cp2_ref[...], W1a, W1b, b1, w2)
    vp_ref[...] = _attn_slab(ep_ref[...], pp0_ref[...], pp1_ref[...],
                             pp2_ref[...], W1a, W1b, b1, w2)
    vd_ref[...] = jnp.sum(drug_ref[...].reshape(VBLK, C, D), axis=1)


_TC1_IN_SPECS = [
    pl.BlockSpec((EBLK, D), lambda i: (i, 0)),
    pl.BlockSpec((EBLK, D), lambda i: (ROW_EMB_P // EBLK + i, 0)),
] + [
    pl.BlockSpec((EBLK, D),
                 lambda i, j=j: (ROW_CPAR // EBLK + j * GRID1 + i, 0))
    for j in range(P)
] + [
    pl.BlockSpec((EBLK, D),
                 lambda i, j=j: (ROW_PPAR // EBLK + j * GRID1 + i, 0))
    for j in range(P)
] + [
    pl.BlockSpec((EBLK, D), lambda i: (ROW_DRUG // EBLK + i, 0)),
    pl.BlockSpec((D, 2 * D), lambda i: (0, 0)),
    pl.BlockSpec((1, D), lambda i: (0, 0)),
    pl.BlockSpec((1, D), lambda i: (0, 0)),
]
_TC1_OUT_SPECS = [
    pl.BlockSpec((VBLK, D), lambda i: (i, 0)),
    pl.BlockSpec((VBLK, D), lambda i: (i, 0)),
    pl.BlockSpec((VBLK, D), lambda i: (i, 0)),
]
_TC1_OUT_SHAPE = [
    jax.ShapeDtypeStruct((B * V, D), jnp.float32),
    jax.ShapeDtypeStruct((B * V, D), jnp.float32),
    jax.ShapeDtypeStruct((B * V, D), jnp.float32),
]


def _seq_body(vc_ref, vp_ref, vd_ref, k_ref,
              Wi_c, Wh_c, bi_c, bh_c, Wi_p, Wh_p, bi_p, bh_p,
              Wi_d, Wh_d, bi_d, bh_d, Kc_ref, Kp_ref, Wfc_ref, bfc_ref,
              out_ref):
    def gru(x, Wi_r, Wh_r, bi_r, bh_r):
        Wi = Wi_r[...]
        Wh = Wh_r[...]
        gi = _dotT(x, Wi) + bi_r[...]
        h = jnp.zeros((B, D), jnp.float32)
        bh = bh_r[...]
        for t in range(V):
            git = gi[t * B:(t + 1) * B]
            gh = _dotT(h, Wh) + bh
            r = jax.nn.sigmoid(git[:, :D] + gh[:, :D])
            z = jax.nn.sigmoid(git[:, D:2 * D] + gh[:, D:2 * D])
            n = jnp.tanh(git[:, 2 * D:] + r * gh[:, 2 * D:])
            h = (1.0 - z) * n + z * h
        return h

    h_c = gru(vc_ref[...], Wi_c, Wh_c, bi_c, bh_c)
    h_p = gru(vp_ref[...], Wi_p, Wh_p, bi_p, bh_p)
    h_d = gru(vd_ref[...], Wi_d, Wh_d, bi_d, bh_d)
    tmp = h_c + h_p + h_d

    krows = k_ref[...]

    def knowledge(rows, K_r):
        kp = _dotT(rows, K_r[...])
        kp3 = kp.reshape(B, C * P, D)
        w = jnp.sum(kp3 * tmp.reshape(B, 1, D), axis=2)
        m = jnp.max(w, axis=1, keepdims=True)
        e = jnp.exp(w - m)
        a = e / jnp.sum(e, axis=1, keepdims=True)
        return jnp.sum(a[:, :, None] * kp3, axis=1)

    k_c = knowledge(krows[:NK], Kc_ref)
    k_p = knowledge(krows[NK:], Kp_ref)
    patient = jnp.concatenate([h_c, h_p, h_d, k_c, k_p], axis=1)
    out_ref[...] = _dotT(patient, Wfc_ref[...]) + bfc_ref[...]


_TC2_IN_SPECS = [
    pl.BlockSpec((B * V, D), lambda i: (0, 0)),
    pl.BlockSpec((B * V, D), lambda i: (0, 0)),
    pl.BlockSpec((B * V, D), lambda i: (0, 0)),
    pl.BlockSpec((2 * NK, D), lambda i: (ROW_KC // (2 * NK), 0)),
] + [pl.BlockSpec((3 * D, D), lambda i: (0, 0)),
     pl.BlockSpec((3 * D, D), lambda i: (0, 0)),
     pl.BlockSpec((1, 3 * D), lambda i: (0, 0)),
     pl.BlockSpec((1, 3 * D), lambda i: (0, 0)),
     ] * 3 + [
    pl.BlockSpec((D, D), lambda i: (0, 0)),
    pl.BlockSpec((D, D), lambda i: (0, 0)),
    pl.BlockSpec((D, 5 * D), lambda i: (0, 0)),
    pl.BlockSpec((1, D), lambda i: (0, 0)),
]
_TC2_OUT_SPECS = pl.BlockSpec((B, D), lambda i: (0, 0))
_TC2_OUT_SHAPE = jax.ShapeDtypeStruct((B, D), jnp.float32)


def _flat_indices(cond_codes, cond_parents, proc_codes, proc_parents,
                  drug_codes, cond_last_parents, proc_last_parents):
    s1 = cond_codes.transpose(1, 0, 2).reshape(NWORK, -1)
    s2 = proc_codes.transpose(1, 0, 2).reshape(NWORK, -1)
    s3 = cond_parents.transpose(3, 1, 0, 2).reshape(NWORK, -1)
    s4 = proc_parents.transpose(3, 1, 0, 2).reshape(NWORK, -1)
    s5 = drug_codes.transpose(1, 0, 2).reshape(NWORK, -1)
    s6 = cond_last_parents.reshape(NWORK, -1)
    s7 = proc_last_parents.reshape(NWORK, -1)
    idx = jnp.concatenate([s1, s2, s3, s4, s5, s6, s7], axis=1)
    return idx.astype(jnp.int32).reshape(NWORK, IPW // 128, 128)


def kernel(cond_codes, cond_parents, proc_codes, proc_parents, drug_codes,
           cond_last_parents, proc_last_parents, E_cond, E_cond_parent,
           E_proc, E_proc_parent, E_drug, W1, b1, W2, K_cond, K_proc,
           Wi_cond, Wh_cond, bi_cond, bh_cond, Wi_proc, Wh_proc, bi_proc,
           bh_proc, Wi_drug, Wh_drug, bi_drug, bh_drug, W_fc, b_fc):
    idx3 = _flat_indices(cond_codes, cond_parents, proc_codes, proc_parents,
                         drug_codes, cond_last_parents, proc_last_parents)
    G = _make_sc_gather()(E_cond, E_proc, E_cond_parent, E_proc_parent,
                          E_drug, idx3)

    vc, vp, vd = pl.pallas_call(
        _attn_body,
        grid=(GRID1,),
        in_specs=_TC1_IN_SPECS,
        out_specs=_TC1_OUT_SPECS,
        out_shape=_TC1_OUT_SHAPE,
    )(G, G, G, G, G, G, G, G, G, W1, b1.reshape(1, D), W2)

    out = pl.pallas_call(
        _seq_body,
        grid=(1,),
        in_specs=_TC2_IN_SPECS,
        out_specs=_TC2_OUT_SPECS,
        out_shape=_TC2_OUT_SHAPE,
    )(vc, vp, vd, G,
      Wi_cond, Wh_cond, bi_cond.reshape(1, 3 * D), bh_cond.reshape(1, 3 * D),
      Wi_proc, Wh_proc, bi_proc.reshape(1, 3 * D), bh_proc.reshape(1, 3 * D),
      Wi_drug, Wh_drug, bi_drug.reshape(1, 3 * D), bh_drug.reshape(1, 3 * D),
      K_cond, K_proc, W_fc, b_fc.reshape(1, D))
    return out
